# baseline (speedup 1.0000x reference)
import jax
import jax.numpy as jnp
from jax import lax
from jax.experimental import pallas as pl
from jax.experimental.pallas import tpu as pltpu
from jax.experimental.pallas import tpu_sc as plsc

_DEPTH = 3
_K = 10
_NC, _NS = 2, 16
_NW = _NC * _NS
_BPB = 8


def _dot(a, b):
    return lax.dot_general(a, b, (((1,), (0,)), ((), ())),
                           preferred_element_type=jnp.float32)


def _tc1(ia_ref, ib_ref, Wemb_ref, Wu2a_ref, Wnb_ref, Wu2b_ref, bU2_ref,
         af_ref, a2_ref, b1_ref, b2t_ref):
    i = pl.program_id(0)
    n_real = pl.num_programs(0) - 1

    @pl.when(i < n_real)
    def _():
        for m in range(af_ref.shape[0]):
            af = jax.nn.relu(_dot(ia_ref[m], Wemb_ref[...]))
            af_ref[m] = af
            a2_ref[m] = _dot(af, Wu2a_ref[...])
            b1_ref[m] = _dot(ib_ref[m], Wnb_ref[...])
            b2t_ref[m] = _dot(ib_ref[m], Wu2b_ref[...]) + bU2_ref[...]

    @pl.when(i == n_real)
    def _():
        af_ref[...] = jnp.zeros_like(af_ref)
        a2_ref[...] = jnp.zeros_like(a2_ref)
        b1_ref[...] = jnp.zeros_like(b1_ref)
        b2t_ref[...] = jnp.zeros_like(b2t_ref)


def _tc2(af_ref, nl_ref, Wu1a_ref, Wu1b_ref, bU1_ref, Wu2a_ref,
         afn_ref, a2n_ref):
    i = pl.program_id(0)
    n_real = pl.num_programs(0) - 1

    @pl.when(i < n_real)
    def _():
        for m in range(afn_ref.shape[0]):
            afn = jax.nn.relu(_dot(af_ref[m], Wu1a_ref[...])
                              + _dot(nl_ref[m], Wu1b_ref[...])
                              + bU1_ref[...])
            afn_ref[m] = afn
            a2n_ref[m] = _dot(afn, Wu2a_ref[...])

    @pl.when(i == n_real)
    def _():
        afn_ref[...] = jnp.zeros_like(afn_ref)
        a2n_ref[...] = jnp.zeros_like(a2n_ref)


def _tc3(af_ref, nl_ref, Wu1a_ref, Wu1b_ref, bU1_ref, Wna_ref, Wself_ref,
         nm_ref, fs1_ref, fs2n_ref):
    i = pl.program_id(0)
    n_real = pl.num_programs(0) - 1

    @pl.when(i < n_real)
    def _():
        for m in range(fs1_ref.shape[0]):
            af2 = jax.nn.relu(_dot(af_ref[m], Wu1a_ref[...])
                              + _dot(nl_ref[m], Wu1b_ref[...])
                              + bU1_ref[...])
            fs1_ref[m] = _dot(af2, Wna_ref[...])
            fs2n_ref[m] = _dot(af2, Wself_ref[...]) * nm_ref[m]

    @pl.when(i == n_real)
    def _():
        fs1_ref[...] = jnp.zeros_like(fs1_ref)
        fs2n_ref[...] = jnp.zeros_like(fs2n_ref)


def _sc_segsum(tabA_ref, tabB_ref, idxa_ref, idxb_ref, nl_ref,
               ia_v, ib_v, rows_a, rows_b, nl_mol, sem_a, sem_b):
    nblk = ia_v.shape[0]
    mpt = idxa_ref.shape[0] // _NW
    wid = lax.axis_index("s") * _NC + lax.axis_index("c")
    for mi in range(mpt):
        m = wid * mpt + mi
        pltpu.sync_copy(idxa_ref.at[m], ia_v)
        pltpu.sync_copy(idxb_ref.at[m], ib_v)

        def blk_body(blk, c):
            cpa = pltpu.async_copy(tabA_ref.at[ia_v.at[blk]], rows_a, sem_a)
            cpb = pltpu.async_copy(tabB_ref.at[ib_v.at[blk]], rows_b, sem_b)
            cpa.wait()
            cpb.wait()

            def atom_body(a, c2):
                r0 = a * _K
                for h in range(8):
                    hs = pl.ds(h * 16, 16)
                    acc = jnp.zeros((16,), jnp.float32)
                    for k in range(_K):
                        acc = acc + jnp.maximum(
                            rows_a[r0 + k, hs] + rows_b[r0 + k, hs], 0.0)
                    nl_mol[blk * _BPB + a, hs] = acc
                return c2

            lax.fori_loop(0, _BPB, atom_body, 0)
            return c

        lax.fori_loop(0, nblk, blk_body, 0)
        pltpu.sync_copy(nl_mol, nl_ref.at[m])


def _sc_final(tabA_ref, tabB_ref, idxa_ref, idxb_ref, fs2_ref,
              outk_ref, ms_ref,
              ia_v, ib_v, rows_a, rows_b, fs2_mol, out_mol, ms_v,
              sem_a, sem_b):
    nblk = ia_v.shape[0]
    mpt = idxa_ref.shape[0] // _NW
    wid = lax.axis_index("s") * _NC + lax.axis_index("c")
    for mi in range(mpt):
        m = wid * mpt + mi
        pltpu.sync_copy(idxa_ref.at[m], ia_v)
        pltpu.sync_copy(idxb_ref.at[m], ib_v)
        pltpu.sync_copy(fs2_ref.at[m], fs2_mol)
        for h in range(8):
            ms_v[pl.ds(h * 16, 16)] = jnp.zeros((16,), jnp.float32)

        def blk_body(blk, c):
            cpa = pltpu.async_copy(tabA_ref.at[ia_v.at[blk]], rows_a, sem_a)
            cpb = pltpu.async_copy(tabB_ref.at[ib_v.at[blk]], rows_b, sem_b)
            cpa.wait()
            cpb.wait()

            def atom_body(a, c2):
                r0 = a * _K
                row = blk * _BPB + a
                for h in range(8):
                    hs = pl.ds(h * 16, 16)
                    acc = jnp.zeros((16,), jnp.float32)
                    for k in range(_K):
                        acc = acc + rows_a[r0 + k, hs] * rows_b[r0 + k, hs]
                    o = acc * fs2_mol[row, hs]
                    out_mol[row, hs] = o
                    ms_v[hs] = ms_v[hs] + o
                return c2

            lax.fori_loop(0, _BPB, atom_body, 0)
            return c

        lax.fori_loop(0, nblk, blk_body, 0)
        pltpu.sync_copy(out_mol, outk_ref.at[m])
        pltpu.sync_copy(ms_v, ms_ref.at[m])


@jax.jit
def kernel(input_atom, input_bond, atom_graph, bond_graph, num_nbs,
           node_mask, W_emb, W_nei_atom, W_nei_bond, W_self, W_U2, b_U2,
           W_U1, b_U1):
    B, N, AF = input_atom.shape
    NB, BF = input_bond.shape[1], input_bond.shape[2]
    H = W_emb.shape[1]
    K = _K
    MB = 4
    G = B // MB + 1
    BP = G * MB
    nblk = N // _BPB

    Wu2a, Wu2b = W_U2[:H], W_U2[H:]
    Wu1a, Wu1b = W_U1[:H], W_U1[H:]
    bU2r = b_U2.reshape(1, H)
    bU1r = b_U1.reshape(1, H)

    valid = jnp.arange(K)[None, None, :] < num_nbs[:, :, None]
    ga = atom_graph[..., 0] * N + atom_graph[..., 1]
    ga = jnp.where(valid, ga, B * N).astype(jnp.int32).reshape(B, nblk, _BPB * K)
    gb = bond_graph[..., 0] * NB + bond_graph[..., 1]
    gb = jnp.where(valid, gb, B * NB).astype(jnp.int32).reshape(B, nblk, _BPB * K)

    full = lambda *shape: pl.BlockSpec(shape, lambda b: (0,) * len(shape))
    inb = lambda *shape: pl.BlockSpec(
        (MB,) + shape, lambda b: (jnp.minimum(b, G - 2),) + (0,) * len(shape))
    outb = lambda *shape: pl.BlockSpec((MB,) + shape,
                                       lambda b: (b,) + (0,) * len(shape))

    af0, a2_0, b1, b2t = pl.pallas_call(
        _tc1,
        grid=(G,),
        in_specs=[inb(N, AF), inb(NB, BF), full(AF, H), full(H, H),
                  full(BF, H), full(BF, H), full(1, H)],
        out_specs=[outb(N, H), outb(N, H), outb(NB, H), outb(NB, H)],
        out_shape=[
            jax.ShapeDtypeStruct((BP, N, H), jnp.float32),
            jax.ShapeDtypeStruct((BP, N, H), jnp.float32),
            jax.ShapeDtypeStruct((BP, NB, H), jnp.float32),
            jax.ShapeDtypeStruct((BP, NB, H), jnp.float32),
        ],
    )(input_atom, input_bond, W_emb, Wu2a, W_nei_bond, Wu2b, bU2r)

    mesh = plsc.VectorSubcoreMesh(core_axis_name="c", subcore_axis_name="s")
    sc_segsum = pl.kernel(
        _sc_segsum,
        out_type=jax.ShapeDtypeStruct((B, N, H), jnp.float32),
        mesh=mesh,
        scratch_types=[
            pltpu.VMEM((nblk, _BPB * K), jnp.int32),
            pltpu.VMEM((nblk, _BPB * K), jnp.int32),
            pltpu.VMEM((_BPB * K, H), jnp.float32),
            pltpu.VMEM((_BPB * K, H), jnp.float32),
            pltpu.VMEM((N, H), jnp.float32),
            pltpu.SemaphoreType.DMA,
            pltpu.SemaphoreType.DMA,
        ],
    )

    tc2 = lambda af, nl: pl.pallas_call(
        _tc2,
        grid=(G,),
        in_specs=[outb(N, H), inb(N, H), full(H, H), full(H, H), full(1, H),
                  full(H, H)],
        out_specs=[outb(N, H), outb(N, H)],
        out_shape=[
            jax.ShapeDtypeStruct((BP, N, H), jnp.float32),
            jax.ShapeDtypeStruct((BP, N, H), jnp.float32),
        ],
    )(af, nl, Wu1a, Wu1b, bU1r, Wu2a)

    af, a2 = af0, a2_0
    for _ in range(_DEPTH - 2):
        nl = sc_segsum(a2.reshape(BP * N, H), b2t.reshape(BP * NB, H), ga, gb)
        af, a2 = tc2(af, nl)
    nl = sc_segsum(a2.reshape(BP * N, H), b2t.reshape(BP * NB, H), ga, gb)

    fs1, fs2n = pl.pallas_call(
        _tc3,
        grid=(G,),
        in_specs=[outb(N, H), inb(N, H), full(H, H), full(H, H), full(1, H),
                  full(H, H), full(H, H), inb(N, 1)],
        out_specs=[outb(N, H), outb(N, H)],
        out_shape=[
            jax.ShapeDtypeStruct((BP, N, H), jnp.float32),
            jax.ShapeDtypeStruct((BP, N, H), jnp.float32),
        ],
    )(af, nl, Wu1a, Wu1b, bU1r, W_nei_atom, W_self, node_mask)

    outk, ms = pl.kernel(
        _sc_final,
        out_type=[
            jax.ShapeDtypeStruct((B, N, H), jnp.float32),
            jax.ShapeDtypeStruct((B, H), jnp.float32),
        ],
        mesh=mesh,
        scratch_types=[
            pltpu.VMEM((nblk, _BPB * K), jnp.int32),
            pltpu.VMEM((nblk, _BPB * K), jnp.int32),
            pltpu.VMEM((_BPB * K, H), jnp.float32),
            pltpu.VMEM((_BPB * K, H), jnp.float32),
            pltpu.VMEM((N, H), jnp.float32),
            pltpu.VMEM((N, H), jnp.float32),
            pltpu.VMEM((H,), jnp.float32),
            pltpu.SemaphoreType.DMA,
            pltpu.SemaphoreType.DMA,
        ],
    )(fs1.reshape(BP * N, H), b1.reshape(BP * NB, H), ga, gb, fs2n)

    return outk, ms
